# baseline (speedup 1.0000x reference)
import functools

import jax
import jax.numpy as jnp
from jax import lax
from jax.experimental import pallas as pl
from jax.experimental.pallas import tpu as pltpu
from jax.experimental.pallas import tpu_sc as plsc

N = 3200000
NUM_SEGMENTS = 16
NC, NS, L = 2, 16, 16
NW = NC * NS
ROWS_PER_W = N // NW
CH = 10000
NCHUNK = ROWS_PER_W // CH
GROUPS = (3 * CH) // 48
UNROLL = 25
ACC = NUM_SEGMENTS * 3 * L

_mesh = plsc.VectorSubcoreMesh(core_axis_name="c", subcore_axis_name="s")


@functools.partial(
    pl.kernel,
    mesh=_mesh,
    out_type=jax.ShapeDtypeStruct((NW, ACC), jnp.float32),
    scratch_types=[
        pltpu.VMEM((3 * CH,), jnp.float32),
        pltpu.VMEM((CH,), jnp.int32),
        pltpu.VMEM((L,), jnp.int32),
        pltpu.VMEM((L,), jnp.int32),
        pltpu.VMEM((ACC,), jnp.float32),
    ],
)
def _seg_max_sc(pos_hbm, ids_hbm, out_hbm, data_v, ids_v, fid_v, lid_v, acc_v):
    wid = lax.axis_index("s") * NC + lax.axis_index("c")
    neg_inf = jnp.full((L,), -jnp.inf, dtype=jnp.float32)
    iota16 = lax.broadcasted_iota(jnp.int32, (L,), 0)

    for i in range(ACC // L):
        acc_v[pl.ds(i * L, L)] = neg_inf

    def chunk_body(t, _):
        r0 = pl.multiple_of(wid * ROWS_PER_W + t * CH, 8)
        pltpu.sync_copy(ids_hbm.at[pl.ds(r0, L)], fid_v)
        pltpu.sync_copy(ids_hbm.at[pl.ds(r0 + CH - L, L)], lid_v)
        pltpu.sync_copy(pos_hbm.at[pl.ds(3 * r0, 3 * CH)], data_v)
        s0 = fid_v[...][0]
        s1 = lid_v[...][L - 1]

        def acc_update(s, rs):
            for j in range(3):
                off = s * (3 * L) + j * L
                acc_v[pl.ds(off, L)] = jnp.maximum(acc_v[pl.ds(off, L)], rs[j])

        @pl.when(s0 == s1)
        def _fast():
            def f_body(it, carry):
                rs = list(carry)
                base = it * (UNROLL * 48)
                for u in range(UNROLL):
                    for j in range(3):
                        v = data_v[pl.ds(base + u * 48 + j * L, L)]
                        rs[j] = jnp.maximum(rs[j], v)
                return tuple(rs)

            rs = lax.fori_loop(0, GROUPS // UNROLL, f_body,
                               (neg_inf, neg_inf, neg_inf))
            acc_update(s0, rs)

        @pl.when(s0 != s1)
        def _slow():
            pltpu.sync_copy(ids_hbm.at[pl.ds(r0, CH)], ids_v)
            nblk = CH // L

            def lower_bound(sv):
                def w_body(_, state):
                    lo, hi = state
                    active = lo < hi
                    mid = (lo + hi) // 2
                    first = ids_v[pl.ds(jnp.minimum(mid, nblk - 1) * L, L)][0]
                    pred = first < sv
                    return (jnp.where(active & pred, mid + 1, lo),
                            jnp.where(active & (~pred), mid, hi))

                kb, _ = lax.fori_loop(0, 10, w_body,
                                      (jnp.int32(0), jnp.int32(nblk)))
                win = ids_v[pl.ds(jnp.maximum(kb - 1, 0) * L, L)]
                cnt = jnp.int32(0)
                for l in range(L):
                    cnt = cnt + (win[l] < sv).astype(jnp.int32)
                return jnp.where(kb == 0, 0, (kb - 1) * L + cnt)

            def seg_body(i, row_lo):
                row_hi = lower_bound(jnp.int32(i + 1))
                a = 3 * row_lo
                b = 3 * row_hi

                def g_body(g, carry):
                    pbase = g * 48
                    rs = []
                    for j in range(3):
                        p = pbase + 16 * j + iota16
                        m = (p >= a) & (p < b)
                        v = data_v[pl.ds(pbase + j * L, L)]
                        rs.append(jnp.maximum(carry[j],
                                              jnp.where(m, v, -jnp.inf)))
                    return tuple(rs)

                rs = lax.fori_loop(a // 48, (b + 47) // 48, g_body,
                                   (neg_inf, neg_inf, neg_inf))
                acc_update(jnp.int32(i), rs)
                return row_hi

            row_lo0 = lower_bound(jnp.int32(0))
            lax.fori_loop(0, NUM_SEGMENTS, seg_body, row_lo0, unroll=False)

        return 0

    lax.fori_loop(0, NCHUNK, chunk_body, 0)
    pltpu.sync_copy(acc_v, out_hbm.at[wid])


def _copy_body(x_ref, o_ref):
    o_ref[...] = x_ref[...]


_IDB = 640000


def _identity_tc(x):
    return pl.pallas_call(
        _copy_body,
        grid=(x.shape[0] // _IDB,),
        in_specs=[pl.BlockSpec((_IDB,), lambda i: (i,))],
        out_specs=pl.BlockSpec((_IDB,), lambda i: (i,)),
        out_shape=jax.ShapeDtypeStruct(x.shape, x.dtype),
    )(x)


def _merge_body(parts_ref, out_ref):
    m = jnp.max(parts_ref[...], axis=0, keepdims=True)
    lane = lax.broadcasted_iota(jnp.int32, (NUM_SEGMENTS, ACC), 1)
    srow = lax.broadcasted_iota(jnp.int32, (NUM_SEGMENTS, ACC), 0)
    seg_ok = (lane // (3 * L)) == srow
    mb = jnp.broadcast_to(m, (NUM_SEGMENTS, ACC))
    cols = []
    for c in range(3):
        sel = jnp.where(seg_ok & (lane % 3 == c), mb, -jnp.inf)
        cols.append(jnp.max(sel, axis=1, keepdims=True))
    cols.append(jnp.zeros((NUM_SEGMENTS, 128 - 3), jnp.float32))
    out_ref[...] = jnp.concatenate(cols, axis=1)


def kernel(pos, batch, W, b):
    del W, b
    posf = _identity_tc(pos.reshape(-1))
    ids = batch.astype(jnp.int32)
    parts = _seg_max_sc(posf, ids)
    padded = pl.pallas_call(
        _merge_body,
        out_shape=jax.ShapeDtypeStruct((NUM_SEGMENTS, 128), jnp.float32),
    )(parts)
    x = padded[:, :3]
    new_pos = jnp.zeros((x.shape[0], 6), dtype=pos.dtype)
    new_batch = jnp.arange(x.shape[0], dtype=jnp.int64)
    return (x, new_pos, new_batch)


# speedup vs baseline: 1.0027x; 1.0027x over previous
import functools

import jax
import jax.numpy as jnp
from jax import lax
from jax.experimental import pallas as pl
from jax.experimental.pallas import tpu as pltpu
from jax.experimental.pallas import tpu_sc as plsc

N = 3200000
NUM_SEGMENTS = 16
NC, NS, L = 2, 16, 16
NW = NC * NS
ROWS_PER_W = N // NW
CH = 10000
NCHUNK = ROWS_PER_W // CH
GROUPS = (3 * CH) // 48
UNROLL = 25
ACC = NUM_SEGMENTS * 3 * L

_mesh = plsc.VectorSubcoreMesh(core_axis_name="c", subcore_axis_name="s")


@functools.partial(
    pl.kernel,
    mesh=_mesh,
    out_type=jax.ShapeDtypeStruct((NW, ACC), jnp.float32),
    scratch_types=[
        pltpu.VMEM((3 * CH,), jnp.float32),
        pltpu.VMEM((CH,), jnp.int32),
        pltpu.VMEM((L,), jnp.int32),
        pltpu.VMEM((L,), jnp.int32),
        pltpu.VMEM((ACC,), jnp.float32),
    ],
)
def _seg_max_sc(pos_hbm, ids_hbm, out_hbm, data_v, ids_v, fid_v, lid_v, acc_v):
    wid = lax.axis_index("s") * NC + lax.axis_index("c")
    neg_inf = jnp.full((L,), -jnp.inf, dtype=jnp.float32)
    iota16 = lax.broadcasted_iota(jnp.int32, (L,), 0)

    for i in range(ACC // L):
        acc_v[pl.ds(i * L, L)] = neg_inf

    def chunk_body(t, _):
        r0 = pl.multiple_of(wid * ROWS_PER_W + t * CH, 8)
        pltpu.sync_copy(ids_hbm.at[pl.ds(r0, L)], fid_v)
        pltpu.sync_copy(ids_hbm.at[pl.ds(r0 + CH - L, L)], lid_v)
        pltpu.sync_copy(pos_hbm.at[pl.ds(3 * r0, 3 * CH)], data_v)
        s0 = fid_v[...][0]
        s1 = lid_v[...][L - 1]

        def acc_update(s, rs):
            for j in range(3):
                off = s * (3 * L) + j * L
                acc_v[pl.ds(off, L)] = jnp.maximum(acc_v[pl.ds(off, L)], rs[j])

        @pl.when(s0 == s1)
        def _fast():
            def f_body(it, carry):
                rs = list(carry)
                base = it * (UNROLL * 48)
                for u in range(UNROLL):
                    for j in range(3):
                        v = data_v[pl.ds(base + u * 48 + j * L, L)]
                        rs[j] = jnp.maximum(rs[j], v)
                return tuple(rs)

            rs = lax.fori_loop(0, GROUPS // UNROLL, f_body,
                               (neg_inf, neg_inf, neg_inf))
            acc_update(s0, rs)

        @pl.when(s0 != s1)
        def _slow():
            pltpu.sync_copy(ids_hbm.at[pl.ds(r0, CH)], ids_v)
            nblk = CH // L

            def lower_bound(sv):
                def w_body(_, state):
                    lo, hi = state
                    active = lo < hi
                    mid = (lo + hi) // 2
                    first = ids_v[pl.ds(jnp.minimum(mid, nblk - 1) * L, L)][0]
                    pred = first < sv
                    return (jnp.where(active & pred, mid + 1, lo),
                            jnp.where(active & (~pred), mid, hi))

                kb, _ = lax.fori_loop(0, 10, w_body,
                                      (jnp.int32(0), jnp.int32(nblk)))
                win = ids_v[pl.ds(jnp.maximum(kb - 1, 0) * L, L)]
                cnt = jnp.int32(0)
                for l in range(L):
                    cnt = cnt + (win[l] < sv).astype(jnp.int32)
                return jnp.where(kb == 0, 0, (kb - 1) * L + cnt)

            def seg_body(i, row_lo):
                row_hi = lower_bound(jnp.int32(i + 1))
                a = 3 * row_lo
                b = 3 * row_hi

                def g_body(g, carry):
                    pbase = g * 48
                    rs = []
                    for j in range(3):
                        p = pbase + 16 * j + iota16
                        m = (p >= a) & (p < b)
                        v = data_v[pl.ds(pbase + j * L, L)]
                        rs.append(jnp.maximum(carry[j],
                                              jnp.where(m, v, -jnp.inf)))
                    return tuple(rs)

                rs = lax.fori_loop(a // 48, (b + 47) // 48, g_body,
                                   (neg_inf, neg_inf, neg_inf))
                acc_update(jnp.int32(i), rs)
                return row_hi

            row_lo0 = lower_bound(jnp.int32(0))
            lax.fori_loop(0, NUM_SEGMENTS, seg_body, row_lo0, unroll=False)

        return 0

    lax.fori_loop(0, NCHUNK, chunk_body, 0)
    pltpu.sync_copy(acc_v, out_hbm.at[wid])


def _copy_body(x_ref, o_ref):
    o_ref[...] = x_ref[...]


_IDB = 640000


def _identity_tc(x):
    return pl.pallas_call(
        _copy_body,
        grid=(x.shape[0] // _IDB,),
        in_specs=[pl.BlockSpec((_IDB,), lambda i: (i,))],
        out_specs=pl.BlockSpec((_IDB,), lambda i: (i,)),
        out_shape=jax.ShapeDtypeStruct(x.shape, x.dtype),
    )(x)


def _merge_body(parts_ref, out_ref):
    m = jnp.max(parts_ref[...], axis=0, keepdims=True)
    lane = lax.broadcasted_iota(jnp.int32, (NUM_SEGMENTS, ACC), 1)
    srow = lax.broadcasted_iota(jnp.int32, (NUM_SEGMENTS, ACC), 0)
    seg_ok = (lane // (3 * L)) == srow
    mb = jnp.broadcast_to(m, (NUM_SEGMENTS, ACC))
    cols = []
    for c in range(3):
        sel = jnp.where(seg_ok & (lane % 3 == c), mb, -jnp.inf)
        cols.append(jnp.max(sel, axis=1, keepdims=True))
    cols.append(jnp.zeros((NUM_SEGMENTS, 128 - 3), jnp.float32))
    out_ref[...] = jnp.concatenate(cols, axis=1)


def kernel(pos, batch, W, b):
    del W, b
    posf = jnp.minimum(pos.reshape(-1), jnp.float32(jnp.inf))
    ids = batch.astype(jnp.int32)
    parts = _seg_max_sc(posf, ids)
    padded = pl.pallas_call(
        _merge_body,
        out_shape=jax.ShapeDtypeStruct((NUM_SEGMENTS, 128), jnp.float32),
    )(parts)
    x = padded[:, :3]
    new_pos = jnp.zeros((x.shape[0], 6), dtype=pos.dtype)
    new_batch = jnp.arange(x.shape[0], dtype=jnp.int64)
    return (x, new_pos, new_batch)


# speedup vs baseline: 6.4616x; 6.4440x over previous
import functools

import jax
import jax.numpy as jnp
from jax import lax
from jax.experimental import pallas as pl
from jax.experimental.pallas import tpu as pltpu
from jax.experimental.pallas import tpu_sc as plsc

N = 3200000
NUM_SEGMENTS = 16
NC, NS, L = 2, 16, 16
NW = NC * NS
ROWS_PER_W = N // NW
CH = 10000
NCHUNK = ROWS_PER_W // CH
GROUPS = CH // L
UNROLL = 25
ACC = NUM_SEGMENTS * 3 * L
RSPLIT = 5120

_mesh = plsc.VectorSubcoreMesh(core_axis_name="c", subcore_axis_name="s")


def _split_body(p_ref, x_ref, y_ref, z_ref):
    t = jnp.transpose(p_ref[...])
    x_ref[...] = t[0]
    y_ref[...] = t[1]
    z_ref[...] = t[2]


def _split_tc(pos):
    plane = jax.ShapeDtypeStruct((N,), jnp.float32)
    return pl.pallas_call(
        _split_body,
        grid=(N // RSPLIT,),
        in_specs=[pl.BlockSpec((RSPLIT, 3), lambda i: (i, 0))],
        out_specs=[pl.BlockSpec((RSPLIT,), lambda i: (i,))] * 3,
        out_shape=[plane, plane, plane],
    )(pos)


@functools.partial(
    pl.kernel,
    mesh=_mesh,
    out_type=jax.ShapeDtypeStruct((NW, ACC), jnp.float32),
    scratch_types=[
        pltpu.VMEM((CH,), jnp.float32),
        pltpu.VMEM((CH,), jnp.float32),
        pltpu.VMEM((CH,), jnp.float32),
        pltpu.VMEM((CH,), jnp.int32),
        pltpu.VMEM((L,), jnp.int32),
        pltpu.VMEM((L,), jnp.int32),
        pltpu.VMEM((ACC,), jnp.float32),
    ],
)
def _seg_max_sc(xs_hbm, ys_hbm, zs_hbm, ids_hbm, out_hbm,
                x_v, y_v, z_v, ids_v, fid_v, lid_v, acc_v):
    wid = lax.axis_index("s") * NC + lax.axis_index("c")
    neg_inf = jnp.full((L,), -jnp.inf, dtype=jnp.float32)
    planes = (x_v, y_v, z_v)

    for i in range(ACC // L):
        acc_v[pl.ds(i * L, L)] = neg_inf

    def chunk_body(t, _):
        r0 = pl.multiple_of(wid * ROWS_PER_W + t * CH, 8)
        pltpu.sync_copy(ids_hbm.at[pl.ds(r0, L)], fid_v)
        pltpu.sync_copy(ids_hbm.at[pl.ds(r0 + CH - L, L)], lid_v)
        pltpu.sync_copy(xs_hbm.at[pl.ds(r0, CH)], x_v)
        pltpu.sync_copy(ys_hbm.at[pl.ds(r0, CH)], y_v)
        pltpu.sync_copy(zs_hbm.at[pl.ds(r0, CH)], z_v)
        s0 = fid_v[...][0]
        s1 = lid_v[...][L - 1]

        def acc_update(s, rs):
            for c in range(3):
                off = s * (3 * L) + c * L
                acc_v[pl.ds(off, L)] = jnp.maximum(acc_v[pl.ds(off, L)], rs[c])

        @pl.when(s0 == s1)
        def _fast():
            def f_body(it, carry):
                rs = list(carry)
                base = it * (UNROLL * L)
                for u in range(UNROLL):
                    for c in range(3):
                        v = planes[c][pl.ds(base + u * L, L)]
                        rs[c] = jnp.maximum(rs[c], v)
                return tuple(rs)

            rs = lax.fori_loop(0, GROUPS // UNROLL, f_body,
                               (neg_inf, neg_inf, neg_inf))
            acc_update(s0, rs)

        @pl.when(s0 != s1)
        def _slow():
            pltpu.sync_copy(ids_hbm.at[pl.ds(r0, CH)], ids_v)

            for s in range(NUM_SEGMENTS):
                @pl.when((s >= s0) & (s <= s1))
                def _sweep(s=s):
                    def g_body(g, carry):
                        id16 = ids_v[pl.ds(g * L, L)]
                        m = id16 == s
                        rs = []
                        for c in range(3):
                            v = planes[c][pl.ds(g * L, L)]
                            rs.append(jnp.maximum(carry[c],
                                                  jnp.where(m, v, -jnp.inf)))
                        return tuple(rs)

                    rs = lax.fori_loop(0, GROUPS, g_body,
                                       (neg_inf, neg_inf, neg_inf))
                    acc_update(s, rs)

        return 0

    lax.fori_loop(0, NCHUNK, chunk_body, 0)
    pltpu.sync_copy(acc_v, out_hbm.at[wid])


def _merge_body(parts_ref, out_ref):
    m = jnp.max(parts_ref[...], axis=0, keepdims=True)
    lane = lax.broadcasted_iota(jnp.int32, (NUM_SEGMENTS, ACC), 1)
    srow = lax.broadcasted_iota(jnp.int32, (NUM_SEGMENTS, ACC), 0)
    seg_ok = (lane // (3 * L)) == srow
    comp = (lane % (3 * L)) // L
    mb = jnp.broadcast_to(m, (NUM_SEGMENTS, ACC))
    cols = []
    for c in range(3):
        sel = jnp.where(seg_ok & (comp == c), mb, -jnp.inf)
        cols.append(jnp.max(sel, axis=1, keepdims=True))
    cols.append(jnp.zeros((NUM_SEGMENTS, 128 - 3), jnp.float32))
    out_ref[...] = jnp.concatenate(cols, axis=1)


def kernel(pos, batch, W, b):
    del W, b
    xs, ys, zs = _split_tc(pos)
    ids = batch.astype(jnp.int32)
    parts = _seg_max_sc(xs, ys, zs, ids)
    padded = pl.pallas_call(
        _merge_body,
        out_shape=jax.ShapeDtypeStruct((NUM_SEGMENTS, 128), jnp.float32),
    )(parts)
    x = padded[:, :3]
    new_pos = jnp.zeros((x.shape[0], 6), dtype=pos.dtype)
    new_batch = jnp.arange(x.shape[0], dtype=jnp.int64)
    return (x, new_pos, new_batch)


# speedup vs baseline: 7.8190x; 1.2101x over previous
import functools

import jax
import jax.numpy as jnp
from jax import lax
from jax.experimental import pallas as pl
from jax.experimental.pallas import tpu as pltpu
from jax.experimental.pallas import tpu_sc as plsc

N = 3200000
NUM_SEGMENTS = 16
NC, NS, L = 2, 16, 16
NW = NC * NS
ROWS_PER_W = N // NW
CH = 10000
NCHUNK = ROWS_PER_W // CH
GROUPS = CH // L
UNROLL = 25
ACC = NUM_SEGMENTS * 3 * L
RSPLIT = 25600

_mesh = plsc.VectorSubcoreMesh(core_axis_name="c", subcore_axis_name="s")


def _split_body(p_ref, x_ref, y_ref, z_ref):
    t = jnp.transpose(p_ref[...])
    x_ref[...] = t[0]
    y_ref[...] = t[1]
    z_ref[...] = t[2]


def _split_tc(pos):
    plane = jax.ShapeDtypeStruct((N,), jnp.float32)
    return pl.pallas_call(
        _split_body,
        grid=(N // RSPLIT,),
        in_specs=[pl.BlockSpec((RSPLIT, 3), lambda i: (i, 0))],
        out_specs=[pl.BlockSpec((RSPLIT,), lambda i: (i,))] * 3,
        out_shape=[plane, plane, plane],
    )(pos)


@functools.partial(
    pl.kernel,
    mesh=_mesh,
    out_type=jax.ShapeDtypeStruct((NW, ACC), jnp.float32),
    scratch_types=[
        pltpu.VMEM((CH,), jnp.float32),
        pltpu.VMEM((CH,), jnp.float32),
        pltpu.VMEM((CH,), jnp.float32),
        pltpu.VMEM((CH,), jnp.int32),
        pltpu.VMEM((L,), jnp.int32),
        pltpu.VMEM((L,), jnp.int32),
        pltpu.VMEM((ACC,), jnp.float32),
    ],
)
def _seg_max_sc(xs_hbm, ys_hbm, zs_hbm, ids_hbm, out_hbm,
                x_v, y_v, z_v, ids_v, fid_v, lid_v, acc_v):
    wid = lax.axis_index("s") * NC + lax.axis_index("c")
    neg_inf = jnp.full((L,), -jnp.inf, dtype=jnp.float32)
    planes = (x_v, y_v, z_v)

    for i in range(ACC // L):
        acc_v[pl.ds(i * L, L)] = neg_inf

    def chunk_body(t, _):
        r0 = pl.multiple_of(wid * ROWS_PER_W + t * CH, 8)
        pltpu.sync_copy(ids_hbm.at[pl.ds(r0, L)], fid_v)
        pltpu.sync_copy(ids_hbm.at[pl.ds(r0 + CH - L, L)], lid_v)
        pltpu.sync_copy(xs_hbm.at[pl.ds(r0, CH)], x_v)
        pltpu.sync_copy(ys_hbm.at[pl.ds(r0, CH)], y_v)
        pltpu.sync_copy(zs_hbm.at[pl.ds(r0, CH)], z_v)
        s0 = fid_v[...][0]
        s1 = lid_v[...][L - 1]

        def acc_update(s, rs):
            for c in range(3):
                off = s * (3 * L) + c * L
                acc_v[pl.ds(off, L)] = jnp.maximum(acc_v[pl.ds(off, L)], rs[c])

        @pl.when(s0 == s1)
        def _fast():
            def f_body(it, carry):
                rs = list(carry)
                base = it * (UNROLL * L)
                for u in range(UNROLL):
                    for c in range(3):
                        v = planes[c][pl.ds(base + u * L, L)]
                        rs[c] = jnp.maximum(rs[c], v)
                return tuple(rs)

            rs = lax.fori_loop(0, GROUPS // UNROLL, f_body,
                               (neg_inf, neg_inf, neg_inf))
            acc_update(s0, rs)

        @pl.when(s0 != s1)
        def _slow():
            pltpu.sync_copy(ids_hbm.at[pl.ds(r0, CH)], ids_v)

            for s in range(NUM_SEGMENTS):
                @pl.when((s >= s0) & (s <= s1))
                def _sweep(s=s):
                    def g_body(g, carry):
                        id16 = ids_v[pl.ds(g * L, L)]
                        m = id16 == s
                        rs = []
                        for c in range(3):
                            v = planes[c][pl.ds(g * L, L)]
                            rs.append(jnp.maximum(carry[c],
                                                  jnp.where(m, v, -jnp.inf)))
                        return tuple(rs)

                    rs = lax.fori_loop(0, GROUPS, g_body,
                                       (neg_inf, neg_inf, neg_inf))
                    acc_update(s, rs)

        return 0

    lax.fori_loop(0, NCHUNK, chunk_body, 0)
    pltpu.sync_copy(acc_v, out_hbm.at[wid])


def _merge_body(parts_ref, out_ref):
    m = jnp.max(parts_ref[...], axis=0, keepdims=True)
    lane = lax.broadcasted_iota(jnp.int32, (NUM_SEGMENTS, ACC), 1)
    srow = lax.broadcasted_iota(jnp.int32, (NUM_SEGMENTS, ACC), 0)
    seg_ok = (lane // (3 * L)) == srow
    comp = (lane % (3 * L)) // L
    mb = jnp.broadcast_to(m, (NUM_SEGMENTS, ACC))
    cols = []
    for c in range(3):
        sel = jnp.where(seg_ok & (comp == c), mb, -jnp.inf)
        cols.append(jnp.max(sel, axis=1, keepdims=True))
    cols.append(jnp.zeros((NUM_SEGMENTS, 128 - 3), jnp.float32))
    out_ref[...] = jnp.concatenate(cols, axis=1)


def kernel(pos, batch, W, b):
    del W, b
    xs, ys, zs = _split_tc(pos)
    ids = batch.astype(jnp.int32)
    parts = _seg_max_sc(xs, ys, zs, ids)
    padded = pl.pallas_call(
        _merge_body,
        out_shape=jax.ShapeDtypeStruct((NUM_SEGMENTS, 128), jnp.float32),
    )(parts)
    x = padded[:, :3]
    new_pos = jnp.zeros((x.shape[0], 6), dtype=pos.dtype)
    new_batch = jnp.arange(x.shape[0], dtype=jnp.int64)
    return (x, new_pos, new_batch)
